# baseline (speedup 1.0000x reference)
import jax
import jax.numpy as jnp
from jax.experimental import pallas as pl
from jax.experimental.pallas import tpu as pltpu

_R = 256
_L = 20


def kernel(x, pos_emb):
    B, T, C = x.shape
    x2 = x.reshape(B * T, C)
    N = (B * T) // _R
    P = T // _R

    def body(x_hbm, pe_hbm, o_hbm, xbuf, pebuf, rsem, psem, wsem):
        def mk_read(i):
            return pltpu.make_async_copy(
                x_hbm.at[pl.ds(i * _R, _R), :], xbuf.at[i], rsem.at[i]
            )

        def mk_write(i):
            return pltpu.make_async_copy(
                xbuf.at[i], o_hbm.at[pl.ds(i * _R, _R), :], wsem.at[i]
            )

        pe_reads = []
        for j in range(P):
            c = pltpu.make_async_copy(
                pe_hbm.at[pl.ds(j * _R, _R), :], pebuf.at[j], psem.at[j]
            )
            c.start()
            pe_reads.append(c)

        reads = {}
        writes = {}
        for i in range(min(_L, N)):
            reads[i] = mk_read(i)
            reads[i].start()

        for i in range(N):
            reads[i].wait()
            if i < P:
                pe_reads[i].wait()
            xbuf[i, :, :] = xbuf[i, :, :] + pebuf[i % P, :, :]
            writes[i] = mk_write(i)
            writes[i].start()
            if i + _L < N:
                reads[i + _L] = mk_read(i + _L)
                reads[i + _L].start()

        for i in range(N):
            writes[i].wait()

    out = pl.pallas_call(
        body,
        in_specs=[
            pl.BlockSpec(memory_space=pltpu.MemorySpace.HBM),
            pl.BlockSpec(memory_space=pltpu.MemorySpace.HBM),
        ],
        out_specs=pl.BlockSpec(memory_space=pltpu.MemorySpace.HBM),
        out_shape=jax.ShapeDtypeStruct((B * T, C), x.dtype),
        scratch_shapes=[
            pltpu.VMEM((N, _R, C), x.dtype),
            pltpu.VMEM((P, _R, C), x.dtype),
            pltpu.SemaphoreType.DMA((N,)),
            pltpu.SemaphoreType.DMA((P,)),
            pltpu.SemaphoreType.DMA((N,)),
        ],
    )(x2, pos_emb)
    return out.reshape(B, T, C)


# speedup vs baseline: 1.0303x; 1.0303x over previous
import jax
import jax.numpy as jnp
from jax.experimental import pallas as pl
from jax.experimental.pallas import tpu as pltpu

_R = 1024
_L = 6


def kernel(x, pos_emb):
    B, T, C = x.shape
    x2 = x.reshape(B * T, C)
    N = (B * T) // _R
    P = T // _R

    def body(x_hbm, pe_hbm, o_hbm, xbuf, pebuf, rsem, psem, wsem):
        def mk_read(i):
            return pltpu.make_async_copy(
                x_hbm.at[pl.ds(i * _R, _R), :], xbuf.at[i], rsem.at[i]
            )

        def mk_write(i):
            return pltpu.make_async_copy(
                xbuf.at[i], o_hbm.at[pl.ds(i * _R, _R), :], wsem.at[i]
            )

        pe_reads = []
        for j in range(P):
            c = pltpu.make_async_copy(
                pe_hbm.at[pl.ds(j * _R, _R), :], pebuf.at[j], psem.at[j]
            )
            c.start()
            pe_reads.append(c)

        reads = {}
        writes = {}
        for i in range(min(_L, N)):
            reads[i] = mk_read(i)
            reads[i].start()

        for i in range(N):
            reads[i].wait()
            if i < P:
                pe_reads[i].wait()
            xbuf[i, :, :] = xbuf[i, :, :] + pebuf[i % P, :, :]
            writes[i] = mk_write(i)
            writes[i].start()
            if i + _L < N:
                reads[i + _L] = mk_read(i + _L)
                reads[i + _L].start()

        for i in range(N):
            writes[i].wait()

    out = pl.pallas_call(
        body,
        in_specs=[
            pl.BlockSpec(memory_space=pltpu.MemorySpace.HBM),
            pl.BlockSpec(memory_space=pltpu.MemorySpace.HBM),
        ],
        out_specs=pl.BlockSpec(memory_space=pltpu.MemorySpace.HBM),
        out_shape=jax.ShapeDtypeStruct((B * T, C), x.dtype),
        scratch_shapes=[
            pltpu.VMEM((N, _R, C), x.dtype),
            pltpu.VMEM((P, _R, C), x.dtype),
            pltpu.SemaphoreType.DMA((N,)),
            pltpu.SemaphoreType.DMA((P,)),
            pltpu.SemaphoreType.DMA((N,)),
        ],
    )(x2, pos_emb)
    return out.reshape(B, T, C)


# speedup vs baseline: 1.0348x; 1.0043x over previous
import jax
import jax.numpy as jnp
from jax.experimental import pallas as pl
from jax.experimental.pallas import tpu as pltpu

_R = 2048
_L = 3


def kernel(x, pos_emb):
    B, T, C = x.shape
    x2 = x.reshape(B * T, C)
    N = (B * T) // _R
    P = T // _R

    def body(x_hbm, pe_hbm, o_hbm, xbuf, pebuf, rsem, psem, wsem):
        def mk_read(i):
            return pltpu.make_async_copy(
                x_hbm.at[pl.ds(i * _R, _R), :], xbuf.at[i], rsem.at[i]
            )

        def mk_write(i):
            return pltpu.make_async_copy(
                xbuf.at[i], o_hbm.at[pl.ds(i * _R, _R), :], wsem.at[i]
            )

        pe_reads = []
        for j in range(P):
            c = pltpu.make_async_copy(
                pe_hbm.at[pl.ds(j * _R, _R), :], pebuf.at[j], psem.at[j]
            )
            c.start()
            pe_reads.append(c)

        reads = {}
        writes = {}
        for i in range(min(_L, N)):
            reads[i] = mk_read(i)
            reads[i].start()

        for i in range(N):
            reads[i].wait()
            if i < P:
                pe_reads[i].wait()
            xbuf[i, :, :] = xbuf[i, :, :] + pebuf[i % P, :, :]
            writes[i] = mk_write(i)
            writes[i].start()
            if i + _L < N:
                reads[i + _L] = mk_read(i + _L)
                reads[i + _L].start()

        for i in range(N):
            writes[i].wait()

    out = pl.pallas_call(
        body,
        in_specs=[
            pl.BlockSpec(memory_space=pltpu.MemorySpace.HBM),
            pl.BlockSpec(memory_space=pltpu.MemorySpace.HBM),
        ],
        out_specs=pl.BlockSpec(memory_space=pltpu.MemorySpace.HBM),
        out_shape=jax.ShapeDtypeStruct((B * T, C), x.dtype),
        scratch_shapes=[
            pltpu.VMEM((N, _R, C), x.dtype),
            pltpu.VMEM((P, _R, C), x.dtype),
            pltpu.SemaphoreType.DMA((N,)),
            pltpu.SemaphoreType.DMA((P,)),
            pltpu.SemaphoreType.DMA((N,)),
        ],
    )(x2, pos_emb)
    return out.reshape(B, T, C)
